# baseline (speedup 1.0000x reference)
import functools

import jax
import jax.numpy as jnp
import numpy as np
from jax import lax
from jax.experimental import pallas as pl
from jax.experimental.pallas import tpu as pltpu
from jax.experimental.pallas import tpu_sc as plsc

N = 2
LQ = 1360
D_MODEL = 256
N_HEADS = 8
N_LEVELS = 4
N_POINTS = 4
OFFSET_DIM = 16
D_STATE = 8
DT_RANK = 1
SHAPES_STATIC = ((32, 32), (16, 16), (8, 8), (4, 4))
LIN = 1360
HLP = N_HEADS * N_LEVELS * N_POINTS
B_ROWS = N * LQ
BQ = 680
N_BLK = B_ROWS // BQ
B_PAD = 3072
BQP = 768
N_BLK_B = B_PAD // BQP
TOTAL_ROWS = 2 * B_ROWS * HLP
_QROWS_L = [h * (w // 2) for (h, w) in SHAPES_STATIC]
_NH_ROWS = sum(_QROWS_L)
_QOFF_L = np.concatenate([[0], np.cumsum(_QROWS_L)[:-1]]).astype(np.int32)


def _prep_constants():
    j = np.arange(HLP)
    lvl = (j // N_POINTS) % N_LEVELS
    head = j // (N_LEVELS * N_POINTS)
    Wl = np.array([w for (_, w) in SHAPES_STATIC], np.float32)[lvl]
    Hl = np.array([h for (h, _) in SHAPES_STATIC], np.float32)[lvl]
    qoff = _QOFF_L[lvl]
    return lvl, head, Wl, Hl, qoff


_LVL, _HEAD, _WL, _HL, _QOFF = _prep_constants()


def _stage_a_body(q_ref, i_ref, wofft_ref, wvalt_ref, bval_ref,
                  off_ref, val_ref):
    off = jnp.dot(q_ref[...], wofft_ref[...], preferred_element_type=jnp.float32)
    off_ref[...] = off * jax.nn.sigmoid(off)
    val_ref[...] = jnp.dot(i_ref[...], wvalt_ref[...],
                           preferred_element_type=jnp.float32) + bval_ref[...]


def _stage_a(query2, inp2, W_offT, W_valT, b_value):
    return pl.pallas_call(
        _stage_a_body,
        grid=(N_BLK,),
        in_specs=[
            pl.BlockSpec((BQ, D_MODEL), lambda i: (i, 0)),
            pl.BlockSpec((BQ, D_MODEL), lambda i: (i, 0)),
            pl.BlockSpec((D_MODEL, HLP * OFFSET_DIM), lambda i: (0, 0)),
            pl.BlockSpec((D_MODEL, D_MODEL), lambda i: (0, 0)),
            pl.BlockSpec((1, D_MODEL), lambda i: (0, 0)),
        ],
        out_specs=[
            pl.BlockSpec((BQ, HLP * OFFSET_DIM), lambda i: (i, 0)),
            pl.BlockSpec((BQ, D_MODEL), lambda i: (i, 0)),
        ],
        out_shape=[
            jax.ShapeDtypeStruct((B_ROWS, HLP * OFFSET_DIM), jnp.float32),
            jax.ShapeDtypeStruct((B_ROWS, D_MODEL), jnp.float32),
        ],
    )(query2, inp2, W_offT, W_valT, b_value)


def _stage_b_body(x_ref, pt_ref, aux_ref, y_ref):
    dtw = aux_ref[:, 0:1]
    dtb = aux_ref[:, 1:2]
    af = aux_ref[:, 2:3]
    et = aux_ref[:, 3:3 + OFFSET_DIM].T
    pt = pt_ref[...]

    def body(t, h):
        xt = x_ref[t]
        proj = jnp.dot(pt, xt, preferred_element_type=jnp.float32)
        be = proj[0:HLP]
        ce = proj[HLP:2 * HLP]
        ue = proj[2 * HLP:3 * HLP]
        dtr = proj[3 * HLP:4 * HLP]
        delta = jax.nn.softplus(dtr * dtw + dtb)
        da = jnp.exp(delta * af)
        h = da * h + delta * be * ue
        y_ref[t] = jnp.dot(et, h * ce, preferred_element_type=jnp.float32)
        return h

    lax.fori_loop(0, HLP, body, jnp.zeros((HLP, BQP), jnp.float32))


def _stage_b(X, PT, AUX):
    return pl.pallas_call(
        _stage_b_body,
        grid=(N_BLK_B,),
        in_specs=[
            pl.BlockSpec((HLP, OFFSET_DIM, BQP), lambda i: (0, 0, i)),
            pl.BlockSpec((4 * HLP, OFFSET_DIM), lambda i: (0, 0)),
            pl.BlockSpec((HLP, 3 + OFFSET_DIM), lambda i: (0, 0)),
        ],
        out_specs=pl.BlockSpec((HLP, OFFSET_DIM, BQP), lambda i: (0, 0, i)),
        out_shape=jax.ShapeDtypeStruct((HLP, OFFSET_DIM, B_PAD), jnp.float32),
    )(X, PT, AUX)


def _stage_c_body(y2_ref, m_ref, bias_ref, h16_ref, refx_ref, refy_ref,
                  nhb_ref, cst_ref, idx_ref, wgt_ref):
    proj = jnp.dot(y2_ref[...], m_ref[...],
                   preferred_element_type=jnp.float32) + bias_ref[...]
    sx = proj[:, 0:HLP]
    sy = proj[:, HLP:2 * HLP]
    aw = proj[:, 2 * HLP:3 * HLP]
    m = jnp.max(aw, axis=1, keepdims=True)
    e = jnp.exp(aw - m)
    denom = jnp.dot(e, h16_ref[...], preferred_element_type=jnp.float32)
    aws = e / denom
    wl = cst_ref[0:1, :]
    hl = cst_ref[1:2, :]
    x = (refx_ref[...] + sx / wl) * wl - 0.5
    y = (refy_ref[...] + sy / hl) * hl - 0.5
    x0 = jnp.floor(x)
    y0 = jnp.floor(y)
    wx1 = x - x0
    wy1 = y - y0
    nhb = nhb_ref[...]
    wq = wl * 0.5
    vx0 = ((x0 >= 0.0) & (x0 <= wl - 1.0)).astype(jnp.float32)
    vx1 = ((x0 + 1.0 >= 0.0) & (x0 + 1.0 <= wl - 1.0)).astype(jnp.float32)
    wx0p = (1.0 - wx1) * vx0
    wx1p = wx1 * vx1
    q = jnp.clip(jnp.floor(x0 * 0.5), 0.0, wq - 1.0)
    k0 = jnp.clip(x0 - 2.0 * q, 0.0, 3.0).astype(jnp.int32)
    k1 = jnp.clip(x0 + 1.0 - 2.0 * q, 0.0, 3.0).astype(jnp.int32)
    qi = q.astype(jnp.int32)
    wqi = wq.astype(jnp.int32)
    for yc in range(2):
        iy = y0 + float(yc)
        vy = ((iy >= 0.0) & (iy <= hl - 1.0)).astype(jnp.float32)
        iyc = jnp.clip(iy, 0.0, hl - 1.0).astype(jnp.int32)
        idx_ref[yc] = nhb + iyc * wqi + qi
        wy = (wy1 if yc else 1.0 - wy1) * vy * aws
        for k in range(4):
            ck = ((k0 == k).astype(jnp.float32) * wx0p
                  + (k1 == k).astype(jnp.float32) * wx1p)
            wgt_ref[yc * 4 + k] = wy * ck


def _stage_c(Y2, M, bias, H16, refx_e, refy_e, nhb, cst):
    return pl.pallas_call(
        _stage_c_body,
        grid=(N_BLK,),
        in_specs=[
            pl.BlockSpec((BQ, HLP * OFFSET_DIM), lambda i: (i, 0)),
            pl.BlockSpec((HLP * OFFSET_DIM, 3 * HLP), lambda i: (0, 0)),
            pl.BlockSpec((1, 3 * HLP), lambda i: (0, 0)),
            pl.BlockSpec((HLP, HLP), lambda i: (0, 0)),
            pl.BlockSpec((BQ, HLP), lambda i: (i, 0)),
            pl.BlockSpec((BQ, HLP), lambda i: (i, 0)),
            pl.BlockSpec((BQ, HLP), lambda i: (i, 0)),
            pl.BlockSpec((8, HLP), lambda i: (0, 0)),
        ],
        out_specs=[
            pl.BlockSpec((2, BQ, HLP), lambda i: (0, i, 0)),
            pl.BlockSpec((8, BQ, HLP), lambda i: (0, i, 0)),
        ],
        out_shape=[
            jax.ShapeDtypeStruct((2, B_ROWS, HLP), jnp.int32),
            jax.ShapeDtypeStruct((8, B_ROWS, HLP), jnp.float32),
        ],
    )(Y2, M, bias, H16, refx_e, refy_e, nhb, cst)


_NW = 32
_ROWS_PER_W = TOTAL_ROWS // _NW
_CHUNK = 320
_NCHUNK = _ROWS_PER_W // _CHUNK


def _gather_rows(table, idx_flat):
    mesh = plsc.VectorSubcoreMesh(core_axis_name="c", subcore_axis_name="s")

    @functools.partial(
        pl.kernel, mesh=mesh,
        out_type=jax.ShapeDtypeStruct((TOTAL_ROWS, 128), jnp.float32),
        scratch_types=[
            pltpu.VMEM((_ROWS_PER_W,), jnp.int32),
            pltpu.VMEM((2, _CHUNK, 128), jnp.float32),
            pltpu.SemaphoreType.DMA,
            pltpu.SemaphoreType.DMA,
            pltpu.SemaphoreType.DMA,
            pltpu.SemaphoreType.DMA,
        ],
    )
    def gather_k(table_hbm, idx_hbm, out_hbm, idx_v, rows_v,
                 sg0, sg1, ss0, ss1):
        wid = lax.axis_index("s") * 2 + lax.axis_index("c")
        base = wid * _ROWS_PER_W
        sg = (sg0, sg1)
        ss = (ss0, ss1)
        pltpu.sync_copy(idx_hbm.at[pl.ds(base, _ROWS_PER_W)], idx_v)
        c_st = [None] * _NCHUNK
        for k in range(_NCHUNK):
            b = k % 2
            if k >= 2:
                c_st[k - 2].wait()
            g = pltpu.async_copy(
                table_hbm.at[idx_v.at[pl.ds(k * _CHUNK, _CHUNK)]],
                rows_v.at[b], sg[b])
            g.wait()
            c_st[k] = pltpu.async_copy(
                rows_v.at[b], out_hbm.at[pl.ds(base + k * _CHUNK, _CHUNK)],
                ss[b])
        for k in range(max(0, _NCHUNK - 2), _NCHUNK):
            c_st[k].wait()

    return gather_k(table, idx_flat)


_BQD = 8


def _stage_d_body(g_ref, wa_ref, wb_ref, a_ref, out_ref):
    s = None
    for yc in range(2):
        g2 = g_ref[yc].reshape(_BQD * HLP, 128)
        w2 = (wa_ref if yc == 0 else wb_ref)[...]
        for k in range(4):
            gc = g2[:, k * 32:(k + 1) * 32] * w2[:, k:k + 1]
            s = gc if s is None else s + gc
    out_ref[...] = jnp.dot(a_ref[...], s, preferred_element_type=jnp.float32)


def _stage_d(G4, w2a, w2b):
    jr = np.arange(_BQD * N_HEADS)
    jq = np.arange(_BQD * HLP)
    ared = jnp.asarray((jq[None, :] // 16 == jr[:, None]).astype(np.float32))
    return pl.pallas_call(
        _stage_d_body,
        grid=(B_ROWS // _BQD,),
        in_specs=[
            pl.BlockSpec((2, _BQD, HLP, 128), lambda i: (0, i, 0, 0)),
            pl.BlockSpec((_BQD * HLP, 4), lambda i: (i, 0)),
            pl.BlockSpec((_BQD * HLP, 4), lambda i: (i, 0)),
            pl.BlockSpec((_BQD * N_HEADS, _BQD * HLP), lambda i: (0, 0)),
        ],
        out_specs=pl.BlockSpec((_BQD * N_HEADS, 32), lambda i: (i, 0)),
        out_shape=jax.ShapeDtypeStruct((B_ROWS * N_HEADS, 32), jnp.float32),
    )(G4, w2a, w2b, ared)


def _stage_e_body(r_ref, woutt_ref, bout_ref, out_ref):
    out_ref[...] = jnp.dot(r_ref[...], woutt_ref[...],
                           preferred_element_type=jnp.float32) + bout_ref[...]


def _stage_e(red2, W_outT, b_out):
    return pl.pallas_call(
        _stage_e_body,
        grid=(N_BLK,),
        in_specs=[
            pl.BlockSpec((BQ, D_MODEL), lambda i: (i, 0)),
            pl.BlockSpec((D_MODEL, D_MODEL), lambda i: (0, 0)),
            pl.BlockSpec((1, D_MODEL), lambda i: (0, 0)),
        ],
        out_specs=pl.BlockSpec((BQ, D_MODEL), lambda i: (i, 0)),
        out_shape=jax.ShapeDtypeStruct((B_ROWS, D_MODEL), jnp.float32),
    )(red2, W_outT, b_out)


def kernel(query, reference_points, input_flatten, input_spatial_shapes,
           input_level_start_index, W_value, b_value, W_off, x_proj_w,
           dt_proj_w, dt_proj_b, A_log, W_so, b_so, W_aw, b_aw, W_out, b_out):
    query2 = query.reshape(B_ROWS, D_MODEL)
    inp2 = input_flatten.reshape(B_ROWS, D_MODEL)
    off, value = _stage_a(query2, inp2, W_off.T, W_value.T,
                          b_value.reshape(1, D_MODEL))
    v4 = value.reshape(N, LIN, N_HEADS, 32)
    parts = []
    pos = 0
    for (Hl, Wl) in SHAPES_STATIC:
        vl = v4[:, pos:pos + Hl * Wl].reshape(
            N, Hl, Wl, N_HEADS, 32).transpose(0, 3, 1, 2, 4)
        vp = jnp.pad(vl, ((0, 0), (0, 0), (0, 0), (0, 3), (0, 0)))
        idx_q = (2 * np.arange(Wl // 2)[:, None]
                 + np.arange(4)[None]).astype(np.int32)
        quads = vp[:, :, :, idx_q, :]
        parts.append(quads.reshape(N, N_HEADS, Hl * (Wl // 2), 128))
        pos += Hl * Wl
    table = jnp.concatenate(parts, axis=2).reshape(
        N * N_HEADS * _NH_ROWS, 128)

    jj = np.arange(HLP)
    w_dt = x_proj_w[0]
    Wb = x_proj_w[1:1 + D_STATE].T
    Wc = x_proj_w[1 + D_STATE:].T
    P = jnp.concatenate([
        Wb[:, jj % D_STATE],
        Wc[:, jj % D_STATE],
        jnp.asarray(np.eye(OFFSET_DIM, dtype=np.float32)[:, jj // D_STATE]),
        jnp.broadcast_to(w_dt[:, None], (OFFSET_DIM, HLP)),
    ], axis=1)
    PT = P.T
    A = -jnp.exp(A_log)
    aux = jnp.concatenate([
        dt_proj_w[:, 0][jj // D_STATE][:, None],
        dt_proj_b[jj // D_STATE][:, None],
        A.reshape(-1)[:, None],
        np.eye(OFFSET_DIM, dtype=np.float32)[jj // D_STATE],
    ], axis=1)

    X = off.reshape(B_ROWS, HLP, OFFSET_DIM).transpose(1, 2, 0)
    X = jnp.pad(X, ((0, 0), (0, 0), (0, B_PAD - B_ROWS)))
    Y = _stage_b(X, PT, aux)
    Y2 = Y.transpose(2, 0, 1)[:B_ROWS].reshape(B_ROWS, HLP * OFFSET_DIM)

    Wso = W_so.reshape(HLP, OFFSET_DIM, 2)
    Waw = W_aw.reshape(HLP, OFFSET_DIM)
    eyeh = np.eye(HLP, dtype=np.float32)
    blk = jnp.einsum('jd,jk->jdk', Wso[:, :, 0], eyeh).reshape(
        HLP * OFFSET_DIM, HLP)
    blky = jnp.einsum('jd,jk->jdk', Wso[:, :, 1], eyeh).reshape(
        HLP * OFFSET_DIM, HLP)
    blka = jnp.einsum('jd,jk->jdk', Waw, eyeh).reshape(HLP * OFFSET_DIM, HLP)
    M = jnp.concatenate([blk, blky, blka], axis=1)
    bias = jnp.concatenate([
        b_so.reshape(HLP, 2)[:, 0], b_so.reshape(HLP, 2)[:, 1],
        b_aw.reshape(HLP)])[None]
    H16 = jnp.asarray(
        (jj[:, None] // OFFSET_DIM == jj[None, :] // OFFSET_DIM)
        .astype(np.float32))
    refp = reference_points.reshape(B_ROWS, N_LEVELS, 2)
    refx_e = refp[:, _LVL, 0]
    refy_e = refp[:, _LVL, 1]
    nhb = ((jnp.arange(B_ROWS)[:, None] // LQ) * N_HEADS
           + jnp.asarray(_HEAD)[None]) * _NH_ROWS + jnp.asarray(_QOFF)[None]
    nhb = nhb.astype(jnp.int32)
    cst = jnp.asarray(np.stack([_WL, _HL] + [np.zeros(HLP, np.float32)] * 6))

    idx, wgt = _stage_c(Y2, M, bias, H16, refx_e, refy_e, nhb, cst)
    G = _gather_rows(table, idx.reshape(-1))
    G4 = G.reshape(2, B_ROWS, HLP, 128)
    wT = wgt.transpose(1, 2, 0).reshape(B_ROWS * HLP, 8)
    red = _stage_d(G4, wT[:, :4], wT[:, 4:])
    out = _stage_e(red.reshape(B_ROWS, D_MODEL), W_out.T,
                   b_out.reshape(1, D_MODEL))
    return out.reshape(N, LQ, D_MODEL)


# speedup vs baseline: 1.0146x; 1.0146x over previous
import functools

import jax
import jax.numpy as jnp
import numpy as np
from jax import lax
from jax.experimental import pallas as pl
from jax.experimental.pallas import tpu as pltpu
from jax.experimental.pallas import tpu_sc as plsc

N = 2
LQ = 1360
D_MODEL = 256
N_HEADS = 8
N_LEVELS = 4
N_POINTS = 4
OFFSET_DIM = 16
D_STATE = 8
DT_RANK = 1
SHAPES_STATIC = ((32, 32), (16, 16), (8, 8), (4, 4))
LIN = 1360
HLP = N_HEADS * N_LEVELS * N_POINTS
B_ROWS = N * LQ
BQ = 680
N_BLK = B_ROWS // BQ
B_PAD = 3072
BQP = 768
N_BLK_B = B_PAD // BQP
TOTAL_ROWS = 2 * B_ROWS * HLP
_QROWS_L = [h * (w // 2) for (h, w) in SHAPES_STATIC]
_NH_ROWS = sum(_QROWS_L)
_QOFF_L = np.concatenate([[0], np.cumsum(_QROWS_L)[:-1]]).astype(np.int32)


def _prep_constants():
    j = np.arange(HLP)
    lvl = (j // N_POINTS) % N_LEVELS
    head = j // (N_LEVELS * N_POINTS)
    Wl = np.array([w for (_, w) in SHAPES_STATIC], np.float32)[lvl]
    Hl = np.array([h for (h, _) in SHAPES_STATIC], np.float32)[lvl]
    qoff = _QOFF_L[lvl]
    return lvl, head, Wl, Hl, qoff


_LVL, _HEAD, _WL, _HL, _QOFF = _prep_constants()


def _stage_a_body(q_ref, i_ref, wofft_ref, wvalt_ref, bval_ref,
                  off_ref, val_ref):
    off = jnp.dot(q_ref[...], wofft_ref[...], preferred_element_type=jnp.float32)
    off_ref[...] = off * jax.nn.sigmoid(off)
    val_ref[...] = jnp.dot(i_ref[...], wvalt_ref[...],
                           preferred_element_type=jnp.float32) + bval_ref[...]


def _stage_a(query2, inp2, W_offT, W_valT, b_value):
    return pl.pallas_call(
        _stage_a_body,
        grid=(N_BLK,),
        in_specs=[
            pl.BlockSpec((BQ, D_MODEL), lambda i: (i, 0)),
            pl.BlockSpec((BQ, D_MODEL), lambda i: (i, 0)),
            pl.BlockSpec((D_MODEL, HLP * OFFSET_DIM), lambda i: (0, 0)),
            pl.BlockSpec((D_MODEL, D_MODEL), lambda i: (0, 0)),
            pl.BlockSpec((1, D_MODEL), lambda i: (0, 0)),
        ],
        out_specs=[
            pl.BlockSpec((BQ, HLP * OFFSET_DIM), lambda i: (i, 0)),
            pl.BlockSpec((BQ, D_MODEL), lambda i: (i, 0)),
        ],
        out_shape=[
            jax.ShapeDtypeStruct((B_ROWS, HLP * OFFSET_DIM), jnp.float32),
            jax.ShapeDtypeStruct((B_ROWS, D_MODEL), jnp.float32),
        ],
    )(query2, inp2, W_offT, W_valT, b_value)


def _stage_b_body(x_ref, pt_ref, aux_ref, y_ref):
    dtw = aux_ref[:, 0:1]
    dtb = aux_ref[:, 1:2]
    af = aux_ref[:, 2:3]
    et = aux_ref[:, 3:3 + OFFSET_DIM].T
    pt = pt_ref[...]

    def body(t, h):
        xt = x_ref[t]
        proj = jnp.dot(pt, xt, preferred_element_type=jnp.float32)
        be = proj[0:HLP]
        ce = proj[HLP:2 * HLP]
        ue = proj[2 * HLP:3 * HLP]
        dtr = proj[3 * HLP:4 * HLP]
        delta = jax.nn.softplus(dtr * dtw + dtb)
        da = jnp.exp(delta * af)
        h = da * h + delta * be * ue
        y_ref[t] = jnp.dot(et, h * ce, preferred_element_type=jnp.float32)
        return h

    lax.fori_loop(0, HLP, body, jnp.zeros((HLP, BQP), jnp.float32))


def _stage_b(X, PT, AUX):
    return pl.pallas_call(
        _stage_b_body,
        grid=(N_BLK_B,),
        in_specs=[
            pl.BlockSpec((HLP, OFFSET_DIM, BQP), lambda i: (0, 0, i)),
            pl.BlockSpec((4 * HLP, OFFSET_DIM), lambda i: (0, 0)),
            pl.BlockSpec((HLP, 3 + OFFSET_DIM), lambda i: (0, 0)),
        ],
        out_specs=pl.BlockSpec((HLP, OFFSET_DIM, BQP), lambda i: (0, 0, i)),
        out_shape=jax.ShapeDtypeStruct((HLP, OFFSET_DIM, B_PAD), jnp.float32),
    )(X, PT, AUX)


def _stage_c_body(y2_ref, m_ref, bias_ref, h16_ref, refx_ref, refy_ref,
                  nhb_ref, cst_ref, idx_ref, wgt_ref):
    proj = jnp.dot(y2_ref[...], m_ref[...],
                   preferred_element_type=jnp.float32) + bias_ref[...]
    sx = proj[:, 0:HLP]
    sy = proj[:, HLP:2 * HLP]
    aw = proj[:, 2 * HLP:3 * HLP]
    m = jnp.max(aw, axis=1, keepdims=True)
    e = jnp.exp(aw - m)
    denom = jnp.dot(e, h16_ref[...], preferred_element_type=jnp.float32)
    aws = e / denom
    wl = cst_ref[0:1, :]
    hl = cst_ref[1:2, :]
    x = (refx_ref[...] + sx / wl) * wl - 0.5
    y = (refy_ref[...] + sy / hl) * hl - 0.5
    x0 = jnp.floor(x)
    y0 = jnp.floor(y)
    wx1 = x - x0
    wy1 = y - y0
    nhb = nhb_ref[...]
    wq = wl * 0.5
    vx0 = ((x0 >= 0.0) & (x0 <= wl - 1.0)).astype(jnp.float32)
    vx1 = ((x0 + 1.0 >= 0.0) & (x0 + 1.0 <= wl - 1.0)).astype(jnp.float32)
    wx0p = (1.0 - wx1) * vx0
    wx1p = wx1 * vx1
    q = jnp.clip(jnp.floor(x0 * 0.5), 0.0, wq - 1.0)
    k0 = jnp.clip(x0 - 2.0 * q, 0.0, 3.0).astype(jnp.int32)
    k1 = jnp.clip(x0 + 1.0 - 2.0 * q, 0.0, 3.0).astype(jnp.int32)
    qi = q.astype(jnp.int32)
    wqi = wq.astype(jnp.int32)
    for yc in range(2):
        iy = y0 + float(yc)
        vy = ((iy >= 0.0) & (iy <= hl - 1.0)).astype(jnp.float32)
        iyc = jnp.clip(iy, 0.0, hl - 1.0).astype(jnp.int32)
        idx_ref[yc] = nhb + iyc * wqi + qi
        wy = (wy1 if yc else 1.0 - wy1) * vy * aws
        for k in range(4):
            ck = ((k0 == k).astype(jnp.float32) * wx0p
                  + (k1 == k).astype(jnp.float32) * wx1p)
            wgt_ref[yc * 4 + k] = wy * ck


def _stage_c(Y2, M, bias, H16, refx_e, refy_e, nhb, cst):
    return pl.pallas_call(
        _stage_c_body,
        grid=(N_BLK,),
        in_specs=[
            pl.BlockSpec((BQ, HLP * OFFSET_DIM), lambda i: (i, 0)),
            pl.BlockSpec((HLP * OFFSET_DIM, 3 * HLP), lambda i: (0, 0)),
            pl.BlockSpec((1, 3 * HLP), lambda i: (0, 0)),
            pl.BlockSpec((HLP, HLP), lambda i: (0, 0)),
            pl.BlockSpec((BQ, HLP), lambda i: (i, 0)),
            pl.BlockSpec((BQ, HLP), lambda i: (i, 0)),
            pl.BlockSpec((BQ, HLP), lambda i: (i, 0)),
            pl.BlockSpec((8, HLP), lambda i: (0, 0)),
        ],
        out_specs=[
            pl.BlockSpec((2, BQ, HLP), lambda i: (0, i, 0)),
            pl.BlockSpec((8, BQ, HLP), lambda i: (0, i, 0)),
        ],
        out_shape=[
            jax.ShapeDtypeStruct((2, B_ROWS, HLP), jnp.int32),
            jax.ShapeDtypeStruct((8, B_ROWS, HLP), jnp.float32),
        ],
    )(Y2, M, bias, H16, refx_e, refy_e, nhb, cst)


_NW = 32
_ROWS_PER_W = TOTAL_ROWS // _NW
_CHUNK = 680
_NCHUNK = _ROWS_PER_W // _CHUNK


def _gather_rows(table, idx_flat):
    mesh = plsc.VectorSubcoreMesh(core_axis_name="c", subcore_axis_name="s")

    @functools.partial(
        pl.kernel, mesh=mesh,
        out_type=jax.ShapeDtypeStruct((TOTAL_ROWS, 128), jnp.float32),
        scratch_types=[
            pltpu.VMEM((_CHUNK,), jnp.int32),
            pltpu.VMEM((_CHUNK, 128), jnp.float32),
            pltpu.SemaphoreType.DMA,
        ],
    )
    def gather_k(table_hbm, idx_hbm, out_hbm, idx_v, rows_v, sem):
        wid = lax.axis_index("s") * 2 + lax.axis_index("c")
        base = wid * _ROWS_PER_W
        for k in range(_NCHUNK):
            b = base + k * _CHUNK
            pltpu.sync_copy(idx_hbm.at[pl.ds(b, _CHUNK)], idx_v)
            pltpu.async_copy(table_hbm.at[idx_v], rows_v, sem).wait()
            pltpu.sync_copy(rows_v, out_hbm.at[pl.ds(b, _CHUNK)])

    return gather_k(table, idx_flat)


_BQD = 8


def _stage_d_body(g_ref, wa_ref, wb_ref, out_ref):
    s = None
    for yc in range(2):
        g2 = g_ref[yc].reshape(_BQD * HLP, 128)
        w2 = (wa_ref if yc == 0 else wb_ref)[...]
        for k in range(4):
            gc = g2[:, k * 32:(k + 1) * 32] * w2[:, k:k + 1]
            s = gc if s is None else s + gc
    out_ref[...] = jnp.sum(s.reshape(_BQD * N_HEADS, 16, 32), axis=1)


def _stage_d(G4, w2a, w2b):
    return pl.pallas_call(
        _stage_d_body,
        grid=(B_ROWS // _BQD,),
        in_specs=[
            pl.BlockSpec((2, _BQD, HLP, 128), lambda i: (0, i, 0, 0)),
            pl.BlockSpec((_BQD * HLP, 4), lambda i: (i, 0)),
            pl.BlockSpec((_BQD * HLP, 4), lambda i: (i, 0)),
        ],
        out_specs=pl.BlockSpec((_BQD * N_HEADS, 32), lambda i: (i, 0)),
        out_shape=jax.ShapeDtypeStruct((B_ROWS * N_HEADS, 32), jnp.float32),
    )(G4, w2a, w2b)


def _stage_e_body(r_ref, woutt_ref, bout_ref, out_ref):
    out_ref[...] = jnp.dot(r_ref[...], woutt_ref[...],
                           preferred_element_type=jnp.float32) + bout_ref[...]


def _stage_e(red2, W_outT, b_out):
    return pl.pallas_call(
        _stage_e_body,
        grid=(N_BLK,),
        in_specs=[
            pl.BlockSpec((BQ, D_MODEL), lambda i: (i, 0)),
            pl.BlockSpec((D_MODEL, D_MODEL), lambda i: (0, 0)),
            pl.BlockSpec((1, D_MODEL), lambda i: (0, 0)),
        ],
        out_specs=pl.BlockSpec((BQ, D_MODEL), lambda i: (i, 0)),
        out_shape=jax.ShapeDtypeStruct((B_ROWS, D_MODEL), jnp.float32),
    )(red2, W_outT, b_out)


def kernel(query, reference_points, input_flatten, input_spatial_shapes,
           input_level_start_index, W_value, b_value, W_off, x_proj_w,
           dt_proj_w, dt_proj_b, A_log, W_so, b_so, W_aw, b_aw, W_out, b_out):
    query2 = query.reshape(B_ROWS, D_MODEL)
    inp2 = input_flatten.reshape(B_ROWS, D_MODEL)
    off, value = _stage_a(query2, inp2, W_off.T, W_value.T,
                          b_value.reshape(1, D_MODEL))
    v4 = value.reshape(N, LIN, N_HEADS, 32)
    parts = []
    pos = 0
    for (Hl, Wl) in SHAPES_STATIC:
        vl = v4[:, pos:pos + Hl * Wl].reshape(
            N, Hl, Wl, N_HEADS, 32).transpose(0, 3, 1, 2, 4)
        vp = jnp.pad(vl, ((0, 0), (0, 0), (0, 0), (0, 3), (0, 0)))
        idx_q = (2 * np.arange(Wl // 2)[:, None]
                 + np.arange(4)[None]).astype(np.int32)
        quads = vp[:, :, :, idx_q, :]
        parts.append(quads.reshape(N, N_HEADS, Hl * (Wl // 2), 128))
        pos += Hl * Wl
    table = jnp.concatenate(parts, axis=2).reshape(
        N * N_HEADS * _NH_ROWS, 128)

    jj = np.arange(HLP)
    w_dt = x_proj_w[0]
    Wb = x_proj_w[1:1 + D_STATE].T
    Wc = x_proj_w[1 + D_STATE:].T
    P = jnp.concatenate([
        Wb[:, jj % D_STATE],
        Wc[:, jj % D_STATE],
        jnp.asarray(np.eye(OFFSET_DIM, dtype=np.float32)[:, jj // D_STATE]),
        jnp.broadcast_to(w_dt[:, None], (OFFSET_DIM, HLP)),
    ], axis=1)
    PT = P.T
    A = -jnp.exp(A_log)
    aux = jnp.concatenate([
        dt_proj_w[:, 0][jj // D_STATE][:, None],
        dt_proj_b[jj // D_STATE][:, None],
        A.reshape(-1)[:, None],
        np.eye(OFFSET_DIM, dtype=np.float32)[jj // D_STATE],
    ], axis=1)

    X = off.reshape(B_ROWS, HLP, OFFSET_DIM).transpose(1, 2, 0)
    X = jnp.pad(X, ((0, 0), (0, 0), (0, B_PAD - B_ROWS)))
    Y = _stage_b(X, PT, aux)
    Y2 = Y.transpose(2, 0, 1)[:B_ROWS].reshape(B_ROWS, HLP * OFFSET_DIM)

    Wso = W_so.reshape(HLP, OFFSET_DIM, 2)
    Waw = W_aw.reshape(HLP, OFFSET_DIM)
    eyeh = np.eye(HLP, dtype=np.float32)
    blk = jnp.einsum('jd,jk->jdk', Wso[:, :, 0], eyeh).reshape(
        HLP * OFFSET_DIM, HLP)
    blky = jnp.einsum('jd,jk->jdk', Wso[:, :, 1], eyeh).reshape(
        HLP * OFFSET_DIM, HLP)
    blka = jnp.einsum('jd,jk->jdk', Waw, eyeh).reshape(HLP * OFFSET_DIM, HLP)
    M = jnp.concatenate([blk, blky, blka], axis=1)
    bias = jnp.concatenate([
        b_so.reshape(HLP, 2)[:, 0], b_so.reshape(HLP, 2)[:, 1],
        b_aw.reshape(HLP)])[None]
    H16 = jnp.asarray(
        (jj[:, None] // OFFSET_DIM == jj[None, :] // OFFSET_DIM)
        .astype(np.float32))
    refp = reference_points.reshape(B_ROWS, N_LEVELS, 2)
    refx_e = refp[:, _LVL, 0]
    refy_e = refp[:, _LVL, 1]
    nhb = ((jnp.arange(B_ROWS)[:, None] // LQ) * N_HEADS
           + jnp.asarray(_HEAD)[None]) * _NH_ROWS + jnp.asarray(_QOFF)[None]
    nhb = nhb.astype(jnp.int32)
    cst = jnp.asarray(np.stack([_WL, _HL] + [np.zeros(HLP, np.float32)] * 6))

    idx, wgt = _stage_c(Y2, M, bias, H16, refx_e, refy_e, nhb, cst)
    G = _gather_rows(table, idx.reshape(-1))
    G4 = G.reshape(2, B_ROWS, HLP, 128)
    wT = wgt.transpose(1, 2, 0).reshape(B_ROWS * HLP, 8)
    red = _stage_d(G4, wT[:, :4], wT[:, 4:])
    out = _stage_e(red.reshape(B_ROWS, D_MODEL), W_out.T,
                   b_out.reshape(1, D_MODEL))
    return out.reshape(N, LQ, D_MODEL)
